# baseline (speedup 1.0000x reference)
import functools

import jax
import jax.numpy as jnp
import numpy as np
from jax import lax
from jax.experimental import pallas as pl
from jax.experimental.pallas import tpu as pltpu
from jax.experimental.pallas import tpu_sc as plsc

_N = 1048576
_GRID = 256
_VOX = np.float32(2.0) / np.float32(255.0)
_MINB = jnp.array([-1.0, -1.0, -1.0], dtype=jnp.float32)
_MAXB = jnp.array([1.0, 1.0, 1.0], dtype=jnp.float32)
_VOXJ = (_MAXB - _MINB) / jnp.array([255.0, 255.0, 255.0], dtype=jnp.float32)
_CORNER_FLAT = np.array([0, 65536, 256, 65792, 1, 65537, 257, 65793],
                        dtype=np.int32)
_OFFSETS = np.array([[0, 0, 0], [1, 0, 0], [0, 1, 0], [1, 1, 0],
                     [0, 0, 1], [1, 0, 1], [0, 1, 1], [1, 1, 1]],
                    dtype=np.float32)

_ROWS = 2048


def _prep_body(pts_ref, vox_ref, idx8_ref, pos_ref):
    p = pts_ref[...]
    t = (p + 1.0) / vox_ref[...]
    c = jnp.floor(t)
    ci = c.astype(jnp.int32)
    base = ci[:, 0:1] * 65536 + ci[:, 1:2] * 256 + ci[:, 2:3]
    j = lax.broadcasted_iota(jnp.int32, (1, 8), 1)
    offc = (j & 1) * 65536 + ((j >> 1) & 1) * 256 + (j >> 2)
    idx8_ref[...] = base + offc
    cx, cy, cz = c[:, 0:1], c[:, 1:2], c[:, 2:3]
    cols = []
    for k in range(8):
        dx, dy, dz = float(k & 1), float((k >> 1) & 1), float(k >> 2)
        cols.append((cx + dx) * _VOX - 1.0)
        cols.append((cy + dy) * _VOX - 1.0)
        cols.append((cz + dz) * _VOX - 1.0)
    pos_ref[...] = jnp.concatenate(cols, axis=1)


def _prep(points):
    n_steps = _N // _ROWS
    return pl.pallas_call(
        _prep_body,
        grid=(n_steps,),
        in_specs=[pl.BlockSpec((_ROWS, 3), lambda i: (i, 0)),
                  pl.BlockSpec((1, 3), lambda i: (0, 0))],
        out_specs=[pl.BlockSpec((_ROWS, 8), lambda i: (i, 0)),
                   pl.BlockSpec((_ROWS, 24), lambda i: (i, 0))],
        out_shape=[jax.ShapeDtypeStruct((_N, 8), jnp.int32),
                   jax.ShapeDtypeStruct((_N, 24), jnp.float32)],
    )(points, _VOXJ.reshape(1, 3))


_NC, _NS = 2, 16
_NW = _NC * _NS
_TOTAL = _N * 8
_PER_W = _TOTAL // _NW
_CH = 2048


def _gather_body(flat_hbm, idx_hbm, out_hbm, idx_v, val_v, sem):
    wid = lax.axis_index("s") * _NC + lax.axis_index("c")
    base = wid * _PER_W

    @pl.loop(0, _PER_W // _CH)
    def _chunk(i):
        off = base + i * _CH
        pltpu.sync_copy(idx_hbm.at[pl.ds(off, _CH)], idx_v)
        pltpu.async_copy(flat_hbm.at[idx_v], val_v, sem).wait()
        pltpu.sync_copy(val_v, out_hbm.at[pl.ds(off, _CH)])


_gather = pl.kernel(
    _gather_body,
    out_type=jax.ShapeDtypeStruct((_TOTAL,), jnp.float32),
    mesh=plsc.VectorSubcoreMesh(core_axis_name="c", subcore_axis_name="s"),
    scratch_types=[
        pltpu.VMEM((_CH,), jnp.int32),
        pltpu.VMEM((_CH,), jnp.float32),
        pltpu.SemaphoreType.DMA,
    ],
)


def kernel(points, sdf_values):
    flat = sdf_values.reshape(-1)
    idx8, pos24 = _prep(points)
    vals = _gather(flat, idx8.reshape(-1))
    return vals.reshape(_N, 8), pos24.reshape(_N, 8, 3)


# speedup vs baseline: 6.2316x; 6.2316x over previous
import jax
import jax.numpy as jnp
import numpy as np
from jax import lax
from jax.experimental import pallas as pl
from jax.experimental.pallas import tpu as pltpu
from jax.experimental.pallas import tpu_sc as plsc

_N = 1048576
_NB = _N // 128
_MINB = jnp.array([-1.0, -1.0, -1.0], dtype=jnp.float32)
_MAXB = jnp.array([1.0, 1.0, 1.0], dtype=jnp.float32)
_VOXJ = (_MAXB - _MINB) / jnp.array([255.0, 255.0, 255.0], dtype=jnp.float32)

_RB = 16


def _prep_body(pts_ref, vox_ref, idx_ref, pos_ref):
    p = pts_ref[...]
    v = vox_ref[...]
    t = (p + 1.0) / v
    c = jnp.floor(t)
    ci = c.astype(jnp.int32)
    base = ci[0] * 65536 + ci[1] * 256 + ci[2]
    k = lax.broadcasted_iota(jnp.int32, (_RB, 8, 128), 1)
    offc = (k & 1) * 65536 + ((k >> 1) & 1) * 256 + (k >> 2)
    idx_ref[...] = base[:, None, :] + offc
    ka = lax.broadcasted_iota(jnp.int32, (3, _RB, 8, 128), 2)
    aa = lax.broadcasted_iota(jnp.int32, (3, _RB, 8, 128), 0)
    d = jnp.where(aa == 0, ka & 1,
                  jnp.where(aa == 1, (ka >> 1) & 1, ka >> 2)).astype(jnp.float32)
    pos_ref[...] = (c[:, :, None, :] + d) * v[:, None, :] - 1.0


def _prep(pts3, voxb):
    n_steps = _NB // _RB
    return pl.pallas_call(
        _prep_body,
        grid=(n_steps,),
        in_specs=[pl.BlockSpec((3, _RB, 128), lambda i: (0, i, 0)),
                  pl.BlockSpec((1, 128), lambda i: (0, 0))],
        out_specs=[pl.BlockSpec((_RB, 8, 128), lambda i: (i, 0, 0)),
                   pl.BlockSpec((3, _RB, 8, 128), lambda i: (0, i, 0, 0))],
        out_shape=[jax.ShapeDtypeStruct((_NB, 8, 128), jnp.int32),
                   jax.ShapeDtypeStruct((3, _NB, 8, 128), jnp.float32)],
    )(pts3, voxb)


_NC, _NS = 2, 16
_NW = _NC * _NS
_TOTAL = _N * 8
_PER_W = _TOTAL // _NW
_CH = 4096
_NCH = _PER_W // _CH


def _gather_body(flat_hbm, idx_hbm, out_hbm,
                 ib0, ib1, vb0, vb1, si0, si1, sg0, sg1, so0, so1):
    wid = lax.axis_index("s") * _NC + lax.axis_index("c")
    base = wid * _PER_W
    ibs, vbs = (ib0, ib1), (vb0, vb1)
    sis, sgs, sos = (si0, si1), (sg0, sg1), (so0, so1)

    def idx_start(c, p):
        pltpu.async_copy(idx_hbm.at[pl.ds(base + c * _CH, _CH)], ibs[p], sis[p])

    def idx_wait(p):
        pltpu.make_async_copy(idx_hbm.at[pl.ds(base, _CH)], ibs[p], sis[p]).wait()

    def gather_start(p):
        pltpu.async_copy(flat_hbm.at[ibs[p]], vbs[p], sgs[p])

    def gather_wait(p):
        pltpu.make_async_copy(flat_hbm.at[ibs[p]], vbs[p], sgs[p]).wait()

    def out_start(c, p):
        pltpu.async_copy(vbs[p], out_hbm.at[pl.ds(base + c * _CH, _CH)], sos[p])

    def out_wait(p):
        pltpu.make_async_copy(vbs[p], out_hbm.at[pl.ds(base, _CH)], sos[p]).wait()

    idx_start(0, 0)
    idx_wait(0)
    idx_start(1, 1)
    gather_start(0)
    gather_wait(0)
    idx_start(2, 0)
    idx_wait(1)
    gather_start(1)
    out_start(0, 0)
    gather_wait(1)
    idx_start(3, 1)
    out_wait(0)
    idx_wait(0)
    gather_start(0)
    out_start(1, 1)

    @pl.loop(2, _NCH - 2, step=2)
    def _steady(c):
        for p in (0, 1):
            cc = c + p
            gather_wait(p)
            idx_start(cc + 2, p)
            out_wait(1 - p)
            idx_wait(1 - p)
            gather_start(1 - p)
            out_start(cc, p)

    gather_wait(0)
    out_wait(1)
    idx_wait(1)
    gather_start(1)
    out_start(_NCH - 2, 0)
    gather_wait(1)
    out_start(_NCH - 1, 1)
    out_wait(0)
    out_wait(1)


_gather = pl.kernel(
    _gather_body,
    out_type=jax.ShapeDtypeStruct((_TOTAL,), jnp.float32),
    mesh=plsc.VectorSubcoreMesh(core_axis_name="c", subcore_axis_name="s"),
    scratch_types=[
        pltpu.VMEM((_CH,), jnp.int32),
        pltpu.VMEM((_CH,), jnp.int32),
        pltpu.VMEM((_CH,), jnp.float32),
        pltpu.VMEM((_CH,), jnp.float32),
        pltpu.SemaphoreType.DMA,
        pltpu.SemaphoreType.DMA,
        pltpu.SemaphoreType.DMA,
        pltpu.SemaphoreType.DMA,
        pltpu.SemaphoreType.DMA,
        pltpu.SemaphoreType.DMA,
    ],
)


def kernel(points, sdf_values):
    flat = sdf_values.reshape(-1)
    pts3 = points.T.reshape(3, _NB, 128)
    voxb = jnp.broadcast_to(_VOXJ[0], (1, 128))
    idx3, pos4 = _prep(pts3, voxb)
    vals = _gather(flat, idx3.reshape(-1))
    corner = vals.reshape(_NB, 8, 128).transpose(0, 2, 1).reshape(_N, 8)
    positions = pos4.transpose(1, 3, 2, 0).reshape(_N, 8, 3)
    return corner, positions


# speedup vs baseline: 8.8408x; 1.4187x over previous
import jax
import jax.numpy as jnp
import numpy as np
from jax import lax
from jax.experimental import pallas as pl
from jax.experimental.pallas import tpu as pltpu
from jax.experimental.pallas import tpu_sc as plsc

_N = 1048576
_NB = _N // 128
_MINB = jnp.array([-1.0, -1.0, -1.0], dtype=jnp.float32)
_MAXB = jnp.array([1.0, 1.0, 1.0], dtype=jnp.float32)
_VOXJ = (_MAXB - _MINB) / jnp.array([255.0, 255.0, 255.0], dtype=jnp.float32)

_RB = 64


def _prep_body(pts_ref, vox_ref, idx_ref, pos_ref):
    p = pts_ref[...]
    v = vox_ref[...]
    t = (p + 1.0) / v
    c = jnp.floor(t)
    ci = c.astype(jnp.int32)
    base = ci[0] * 65536 + ci[1] * 256 + ci[2]
    k = lax.broadcasted_iota(jnp.int32, (1, 8, 128), 1)
    offc = (k & 1) * 65536 + ((k >> 1) & 1) * 256 + (k >> 2)
    idx_ref[...] = base[:, None, :] + offc
    ka = lax.broadcasted_iota(jnp.int32, (3, 1, 8, 128), 2)
    aa = lax.broadcasted_iota(jnp.int32, (3, 1, 8, 128), 0)
    d = jnp.where(aa == 0, ka & 1,
                  jnp.where(aa == 1, (ka >> 1) & 1, ka >> 2)).astype(jnp.float32)
    dvm = d * v[:, None, :] - 1.0
    cv = c * v
    pos_ref[...] = cv[:, :, None, :] + dvm


def _prep(pts3, voxb):
    n_steps = _NB // _RB
    return pl.pallas_call(
        _prep_body,
        grid=(n_steps,),
        in_specs=[pl.BlockSpec((3, _RB, 128), lambda i: (0, i, 0)),
                  pl.BlockSpec((1, 128), lambda i: (0, 0))],
        out_specs=[pl.BlockSpec((_RB, 8, 128), lambda i: (i, 0, 0)),
                   pl.BlockSpec((3, _RB, 8, 128), lambda i: (0, i, 0, 0))],
        out_shape=[jax.ShapeDtypeStruct((_NB, 8, 128), jnp.int32),
                   jax.ShapeDtypeStruct((3, _NB, 8, 128), jnp.float32)],
    )(pts3, voxb)


_NC, _NS = 2, 16
_NW = _NC * _NS
_TOTAL = _N * 8
_PER_W = _TOTAL // _NW
_CH = 8192
_NCH = _PER_W // _CH


def _gather_body(flat_hbm, idx_hbm, out_hbm,
                 ib0, ib1, vb0, vb1, si0, si1, sg0, sg1, so0, so1):
    wid = lax.axis_index("s") * _NC + lax.axis_index("c")
    base = wid * _PER_W
    ibs, vbs = (ib0, ib1), (vb0, vb1)
    sis, sgs, sos = (si0, si1), (sg0, sg1), (so0, so1)

    def idx_start(c, p):
        pltpu.async_copy(idx_hbm.at[pl.ds(base + c * _CH, _CH)], ibs[p], sis[p])

    def idx_wait(p):
        pltpu.make_async_copy(idx_hbm.at[pl.ds(base, _CH)], ibs[p], sis[p]).wait()

    def gather_start(p):
        pltpu.async_copy(flat_hbm.at[ibs[p]], vbs[p], sgs[p])

    def gather_wait(p):
        pltpu.make_async_copy(flat_hbm.at[ibs[p]], vbs[p], sgs[p]).wait()

    def out_start(c, p):
        pltpu.async_copy(vbs[p], out_hbm.at[pl.ds(base + c * _CH, _CH)], sos[p])

    def out_wait(p):
        pltpu.make_async_copy(vbs[p], out_hbm.at[pl.ds(base, _CH)], sos[p]).wait()

    idx_start(0, 0)
    idx_wait(0)
    idx_start(1, 1)
    gather_start(0)
    gather_wait(0)
    idx_start(2, 0)
    idx_wait(1)
    gather_start(1)
    out_start(0, 0)
    gather_wait(1)
    idx_start(3, 1)
    out_wait(0)
    idx_wait(0)
    gather_start(0)
    out_start(1, 1)

    @pl.loop(2, _NCH - 2, step=2)
    def _steady(c):
        for p in (0, 1):
            cc = c + p
            gather_wait(p)
            idx_start(cc + 2, p)
            out_wait(1 - p)
            idx_wait(1 - p)
            gather_start(1 - p)
            out_start(cc, p)

    gather_wait(0)
    out_wait(1)
    idx_wait(1)
    gather_start(1)
    out_start(_NCH - 2, 0)
    gather_wait(1)
    out_start(_NCH - 1, 1)
    out_wait(0)
    out_wait(1)


_gather = pl.kernel(
    _gather_body,
    out_type=jax.ShapeDtypeStruct((_TOTAL,), jnp.float32),
    mesh=plsc.VectorSubcoreMesh(core_axis_name="c", subcore_axis_name="s"),
    scratch_types=[
        pltpu.VMEM((_CH,), jnp.int32),
        pltpu.VMEM((_CH,), jnp.int32),
        pltpu.VMEM((_CH,), jnp.float32),
        pltpu.VMEM((_CH,), jnp.float32),
        pltpu.SemaphoreType.DMA,
        pltpu.SemaphoreType.DMA,
        pltpu.SemaphoreType.DMA,
        pltpu.SemaphoreType.DMA,
        pltpu.SemaphoreType.DMA,
        pltpu.SemaphoreType.DMA,
    ],
)


def kernel(points, sdf_values):
    flat = sdf_values.reshape(-1)
    pts3 = points.T.reshape(3, _NB, 128)
    voxb = jnp.broadcast_to(_VOXJ[0], (1, 128))
    idx3, pos4 = _prep(pts3, voxb)
    vals = _gather(flat, idx3.reshape(-1))
    corner = vals.reshape(_NB, 8, 128).transpose(0, 2, 1).reshape(_N, 8)
    positions = pos4.transpose(1, 3, 2, 0).reshape(_N, 8, 3)
    return corner, positions


# speedup vs baseline: 9.0650x; 1.0254x over previous
import jax
import jax.numpy as jnp
import numpy as np
from jax import lax
from jax.experimental import pallas as pl
from jax.experimental.pallas import tpu as pltpu
from jax.experimental.pallas import tpu_sc as plsc

_N = 1048576
_NB = _N // 128
_MINB = jnp.array([-1.0, -1.0, -1.0], dtype=jnp.float32)
_MAXB = jnp.array([1.0, 1.0, 1.0], dtype=jnp.float32)
_VOXJ = (_MAXB - _MINB) / jnp.array([255.0, 255.0, 255.0], dtype=jnp.float32)

_RB = 64


def _prep_body(pts_ref, vox_ref, idx_ref, pos_ref):
    p = pts_ref[...]
    v = vox_ref[...]
    t = (p + 1.0) / v
    c = jnp.floor(t)
    ci = c.astype(jnp.int32)
    base = ci[0] * 65536 + ci[1] * 256 + ci[2]
    k = lax.broadcasted_iota(jnp.int32, (1, 8, 128), 1)
    offc = (k & 1) * 65536 + ((k >> 1) & 1) * 256 + (k >> 2)
    idx_ref[...] = base[:, None, :] + offc
    ka = lax.broadcasted_iota(jnp.int32, (3, 1, 8, 128), 2)
    aa = lax.broadcasted_iota(jnp.int32, (3, 1, 8, 128), 0)
    d = jnp.where(aa == 0, ka & 1,
                  jnp.where(aa == 1, (ka >> 1) & 1, ka >> 2)).astype(jnp.float32)
    dvm = d * v[:, None, :] - 1.0
    cv = c * v
    pos_ref[...] = cv[:, :, None, :] + dvm


def _prep(pts3, voxb):
    n_steps = _NB // _RB
    return pl.pallas_call(
        _prep_body,
        grid=(n_steps,),
        in_specs=[pl.BlockSpec((3, _RB, 128), lambda i: (0, i, 0)),
                  pl.BlockSpec((1, 128), lambda i: (0, 0))],
        out_specs=[pl.BlockSpec((_RB, 8, 128), lambda i: (i, 0, 0)),
                   pl.BlockSpec((3, _RB, 8, 128), lambda i: (0, i, 0, 0))],
        out_shape=[jax.ShapeDtypeStruct((_NB, 8, 128), jnp.int32),
                   jax.ShapeDtypeStruct((3, _NB, 8, 128), jnp.float32)],
    )(pts3, voxb)


_NC, _NS = 2, 16
_NW = _NC * _NS
_TOTAL = _N * 8
_PER_W = _TOTAL // _NW
_CH = 4096
_NCH = _PER_W // _CH


def _gather_body(flat_hbm, idx_hbm, out_hbm,
                 ib0, ib1, ib2, vb0, vb1, vb2,
                 si0, si1, si2, sg0, sg1, sg2, so0, so1, so2):
    wid = lax.axis_index("s") * _NC + lax.axis_index("c")
    base = wid * _PER_W
    ibs, vbs = (ib0, ib1, ib2), (vb0, vb1, vb2)
    sis, sgs, sos = (si0, si1, si2), (sg0, sg1, sg2), (so0, so1, so2)

    def idx_start(c, p):
        pltpu.async_copy(idx_hbm.at[pl.ds(base + c * _CH, _CH)], ibs[p], sis[p])

    def idx_wait(p):
        pltpu.make_async_copy(idx_hbm.at[pl.ds(base, _CH)], ibs[p], sis[p]).wait()

    def gather_start(p):
        pltpu.async_copy(flat_hbm.at[ibs[p]], vbs[p], sgs[p])

    def gather_wait(p):
        pltpu.make_async_copy(flat_hbm.at[ibs[p]], vbs[p], sgs[p]).wait()

    def out_start(c, p):
        pltpu.async_copy(vbs[p], out_hbm.at[pl.ds(base + c * _CH, _CH)], sos[p])

    def out_wait(p):
        pltpu.make_async_copy(vbs[p], out_hbm.at[pl.ds(base, _CH)], sos[p]).wait()

    idx_start(0, 0)
    idx_start(1, 1)
    idx_start(2, 2)
    idx_wait(0)
    gather_start(0)
    idx_wait(1)
    gather_start(1)
    gather_wait(0)
    idx_start(3, 0)
    out_start(0, 0)
    idx_wait(2)
    gather_start(2)

    @pl.loop(1, _NCH - 3, step=3)
    def _steady(c):
        for j in range(3):
            p = (1 + j) % 3
            p2 = (p + 2) % 3
            cc = c + j
            gather_wait(p)
            idx_start(cc + 3, p)
            out_start(cc, p)
            idx_wait(p2)
            out_wait(p2)
            gather_start(p2)

    gather_wait(1)
    out_start(_NCH - 3, 1)
    idx_wait(0)
    out_wait(0)
    gather_start(0)
    gather_wait(2)
    out_start(_NCH - 2, 2)
    gather_wait(0)
    out_start(_NCH - 1, 0)
    out_wait(1)
    out_wait(2)
    out_wait(0)


_gather = pl.kernel(
    _gather_body,
    out_type=jax.ShapeDtypeStruct((_TOTAL,), jnp.float32),
    mesh=plsc.VectorSubcoreMesh(core_axis_name="c", subcore_axis_name="s"),
    scratch_types=[
        pltpu.VMEM((_CH,), jnp.int32),
        pltpu.VMEM((_CH,), jnp.int32),
        pltpu.VMEM((_CH,), jnp.int32),
        pltpu.VMEM((_CH,), jnp.float32),
        pltpu.VMEM((_CH,), jnp.float32),
        pltpu.VMEM((_CH,), jnp.float32),
        pltpu.SemaphoreType.DMA,
        pltpu.SemaphoreType.DMA,
        pltpu.SemaphoreType.DMA,
        pltpu.SemaphoreType.DMA,
        pltpu.SemaphoreType.DMA,
        pltpu.SemaphoreType.DMA,
        pltpu.SemaphoreType.DMA,
        pltpu.SemaphoreType.DMA,
        pltpu.SemaphoreType.DMA,
    ],
)


def kernel(points, sdf_values):
    flat = sdf_values.reshape(-1)
    pts3 = points.T.reshape(3, _NB, 128)
    voxb = jnp.broadcast_to(_VOXJ[0], (1, 128))
    idx3, pos4 = _prep(pts3, voxb)
    vals = _gather(flat, idx3.reshape(-1))
    corner = vals.reshape(_NB, 8, 128).transpose(0, 2, 1).reshape(_N, 8)
    positions = pos4.transpose(1, 3, 2, 0).reshape(_N, 8, 3)
    return corner, positions
